# baseline (speedup 1.0000x reference)
import functools

import jax
import jax.numpy as jnp
import numpy as np
from jax import lax
from jax.experimental import pallas as pl
from jax.experimental.pallas import tpu as pltpu
from jax.experimental.pallas import tpu_sc as plsc

_NN = {'word': 50000, 'topic': 5000, 'doc': 10000}
_REL = [('ww', 'word', 'word', 262144), ('wt', 'word', 'topic', 65536),
        ('wd', 'word', 'doc', 131072), ('tt', 'topic', 'topic', 32768),
        ('td', 'topic', 'doc', 65536)]
_IDX = {'word': 0, 'topic': 1, 'doc': 2}
_D, _H, _DK = 128, 8, 16
_NW = 32
_B = 128
_EW = 16
_SQRT_DK = float(np.sqrt(_DK))
_NPAD = {'word': 51200, 'topic': 6400, 'doc': 12800}
_BN = {'word': 400, 'topic': 200, 'doc': 400}


def _combine_weights(Wk, Wq, Wv, bk, bq, bv, rel_att, rel_pri, rel_msg):
    def body(wk, wq, wv, bk_r, bq_r, bv_r, ratt, rpri, rmsg,
             mk_o, ck_o, mv_o, cv_o, wqs_o, bqs_o):
        for ei, (et, s, d, ne) in enumerate(_REL):
            i = _IDX[s]
            wkt = wk[i].T
            wvt = wv[i].T
            mk_cols, mv_cols = [], []
            ck_cols, cv_cols = [], []
            for h in range(_H):
                a_att = ratt[ei, h] * rpri[ei, h]
                a_msg = rmsg[ei, h]
                sl = slice(h * _DK, (h + 1) * _DK)
                mk_cols.append(wkt[:, sl] @ a_att)
                mv_cols.append(wvt[:, sl] @ a_msg)
                ck_cols.append(bk_r[i, sl].reshape(1, _DK) @ a_att)
                cv_cols.append(bv_r[i, sl].reshape(1, _DK) @ a_msg)
            mk_o[ei] = jnp.concatenate(mk_cols, axis=1)
            mv_o[ei] = jnp.concatenate(mv_cols, axis=1)
            ck_o[ei] = jnp.concatenate(ck_cols, axis=1).reshape(_D)
            cv_o[ei] = jnp.concatenate(cv_cols, axis=1).reshape(_D)
        inv = 1.0 / _SQRT_DK
        wqs_o[...] = jnp.transpose(wq[...], (0, 2, 1)) * inv
        bqs_o[...] = bq_r[...] * inv

    f32 = jnp.float32
    return pl.pallas_call(
        body,
        out_shape=(
            jax.ShapeDtypeStruct((5, _D, _D), f32),
            jax.ShapeDtypeStruct((5, _D), f32),
            jax.ShapeDtypeStruct((5, _D, _D), f32),
            jax.ShapeDtypeStruct((5, _D), f32),
            jax.ShapeDtypeStruct((3, _D, _D), f32),
            jax.ShapeDtypeStruct((3, _D), f32),
        ),
    )(Wk, Wq, Wv, bk, bq, bv, rel_att, rel_pri, rel_msg)


def _prep_type(x, wc, bc, out_cols):
    n = x.shape[0]
    bn = 1024
    grid = (pl.cdiv(n, bn),)
    ncol_total = wc.shape[1]

    def body(x_ref, wc_ref, bc_ref, *outs):
        y = jnp.dot(x_ref[...], wc_ref[...],
                    preferred_element_type=jnp.float32) + bc_ref[...]
        c0 = 0
        for o, nc in zip(outs, out_cols):
            o[...] = y[:, c0:c0 + nc]
            c0 += nc

    out_shape = tuple(jax.ShapeDtypeStruct((n, nc), jnp.float32)
                      for nc in out_cols)
    out_specs = tuple(pl.BlockSpec((bn, nc), lambda i: (i, 0))
                      for nc in out_cols)
    return pl.pallas_call(
        body,
        grid=grid,
        in_specs=[
            pl.BlockSpec((bn, _D), lambda i: (i, 0)),
            pl.BlockSpec((_D, ncol_total), lambda i: (0, 0)),
            pl.BlockSpec((1, ncol_total), lambda i: (0, 0)),
        ],
        out_specs=out_specs,
        out_shape=out_shape,
    )(x, wc, bc)


def _sc_att(tables):
    f32, i32 = jnp.float32, jnp.int32
    mesh = plsc.VectorSubcoreMesh(core_axis_name="c", subcore_axis_name="s")
    nrel = len(tables)

    def body(*refs):
        ins = refs[:4 * nrel]
        outs = refs[4 * nrel:5 * nrel]
        (idx_s0, idx_d0, kt_v0, qs_v0, att_v0,
         idx_s1, idx_d1, kt_v1, qs_v1, att_v1,
         semk0, semq0, semk1, semq1) = refs[5 * nrel:]
        bufs = ((idx_s0, idx_d0, kt_v0, qs_v0, att_v0, semk0, semq0),
                (idx_s1, idx_d1, kt_v1, qs_v1, att_v1, semk1, semq1))
        cid = lax.axis_index("c")
        sid = lax.axis_index("s")
        wid = sid * 2 + cid
        lastm = lax.iota(i32, 16) == 15

        for r in range(nrel):
            kt, qs, src, dst = ins[4 * r:4 * r + 4]
            att = outs[r]
            ne = tables[r][4]
            epw = ne // _NW
            nblk = epw // _B
            base0 = wid * epw

            def issue(base, buf, kt=kt, qs=qs, src=src, dst=dst):
                idx_s, idx_d, kt_v, qs_v, _av, semk, semq = buf
                pltpu.sync_copy(src.at[pl.ds(base, _B)], idx_s)
                pltpu.sync_copy(dst.at[pl.ds(base, _B)], idx_d)
                pltpu.async_copy(kt.at[idx_s], kt_v, semk)
                pltpu.async_copy(qs.at[idx_d], qs_v, semq)

            def wait(buf, kt=kt, qs=qs):
                _is, _id, kt_v, qs_v, _av, semk, semq = buf
                pltpu.make_async_copy(kt.at[pl.ds(0, _B)], kt_v, semk).wait()
                pltpu.make_async_copy(qs.at[pl.ds(0, _B)], qs_v, semq).wait()

            def compute(base, buf, att=att):
                _is, _id, kt_v, qs_v, att_v, _sk, _sq = buf

                def grp(g, _):
                    eids = lax.iota(i32, 16) + g * 16
                    for h in range(_H):
                        acc = jnp.zeros((16,), f32)
                        for d in range(_DK):
                            col = jnp.full((16,), h * _DK + d, i32)
                            acc = acc + (plsc.load_gather(kt_v, [eids, col]) *
                                         plsc.load_gather(qs_v, [eids, col]))
                        plsc.store_scatter(
                            att_v, [eids, jnp.full((16,), h, i32)], acc)
                    return 0

                lax.fori_loop(0, _B // 16, grp, 0)
                pltpu.sync_copy(att_v, att.at[pl.ds(base, _B)])

            def pair(i, _, base0=base0, nblk=nblk):
                b0 = base0 + (2 * i) * _B
                b1 = base0 + (2 * i + 1) * _B
                wait(bufs[0])
                issue(b1, bufs[1])
                compute(b0, bufs[0])
                wait(bufs[1])

                @pl.when(2 * i + 2 < nblk)
                def _():
                    issue(base0 + (2 * i + 2) * _B, bufs[0])

                compute(b1, bufs[1])
                return 0

            issue(base0, bufs[0])
            lax.fori_loop(0, nblk // 2, pair, 0)

    out_type = tuple(jax.ShapeDtypeStruct((t[4], _H), f32) for t in tables)
    args = []
    for kt, qs, src, dst, ne in tables:
        args += [kt, qs, src, dst]
    return pl.kernel(
        body,
        out_type=out_type,
        mesh=mesh,
        compiler_params=pltpu.CompilerParams(needs_layout_passes=False,
                                             use_tc_tiling_on_sc=False),
        scratch_types=[
            pltpu.VMEM((_B,), i32),
            pltpu.VMEM((_B,), i32),
            pltpu.VMEM((_B, _D), f32),
            pltpu.VMEM((_B, _D), f32),
            pltpu.VMEM((_B, _H), f32),
            pltpu.VMEM((_B,), i32),
            pltpu.VMEM((_B,), i32),
            pltpu.VMEM((_B, _D), f32),
            pltpu.VMEM((_B, _D), f32),
            pltpu.VMEM((_B, _H), f32),
            pltpu.SemaphoreType.DMA,
            pltpu.SemaphoreType.DMA,
            pltpu.SemaphoreType.DMA,
            pltpu.SemaphoreType.DMA,
        ],
    )(*args)


def _max_exp(att):
    e = att.shape[0]
    be = 8192
    grid = (e // be,)

    def mx_body(a_ref, o_ref):
        i = pl.program_id(0)
        m = jnp.max(a_ref[...], axis=0, keepdims=True)

        @pl.when(i == 0)
        def _():
            o_ref[...] = m

        @pl.when(i > 0)
        def _():
            o_ref[...] = jnp.maximum(o_ref[...], m)

    mx = pl.pallas_call(
        mx_body,
        grid=grid,
        in_specs=[pl.BlockSpec((be, _H), lambda i: (i, 0))],
        out_specs=pl.BlockSpec((1, _H), lambda i: (0, 0)),
        out_shape=jax.ShapeDtypeStruct((1, _H), jnp.float32),
    )(att)

    def ex_body(a_ref, m_ref, o_ref):
        ex = jnp.exp(a_ref[...] - m_ref[...])
        o_ref[...] = jnp.concatenate(
            [ex, jnp.zeros((be, _EW - _H), jnp.float32)], axis=1)

    return pl.pallas_call(
        ex_body,
        grid=grid,
        in_specs=[pl.BlockSpec((be, _H), lambda i: (i, 0)),
                  pl.BlockSpec((1, _H), lambda i: (0, 0))],
        out_specs=pl.BlockSpec((be, _EW), lambda i: (i, 0)),
        out_shape=jax.ShapeDtypeStruct((e, _EW), jnp.float32),
    )(att, mx)


def _sc_agg(vt, ex, src, dst, *, ne, npad, ncols, head_lo, with_den,
            do_num=True, blk_sz=_B):
    f32, i32 = jnp.float32, jnp.int32
    mesh = plsc.VectorSubcoreMesh(core_axis_name="c", subcore_axis_name="s")
    nvec = ncols // 16
    rps = npad // 16

    def chunked(total):
        out = []
        off = 0
        while off < total:
            sz = min(blk_sz, total - off)
            out.append((off, sz))
            off += sz
        return out

    def body(vt_r, ex_r, src_r, dst_r, *rest):
        pos = 0
        num_o = den_o = num_sh = den_sh = None
        idx_s = vt_v = w_v = None
        if do_num:
            num_o = rest[pos]
            pos += 1
        if with_den:
            den_o = rest[pos]
            pos += 1
        if do_num:
            num_sh = rest[pos]
            pos += 1
        if with_den:
            den_sh = rest[pos]
            pos += 1
        if do_num:
            idx_s, vt_v, w_v = rest[pos:pos + 3]
            pos += 3
        (idx_d, ex_v, sem) = rest[pos:]
        cid = lax.axis_index("c")
        sid = lax.axis_index("s")
        wid = sid * 2 + cid
        zero = jnp.zeros((16,), f32)

        def zrow(j, _):
            if do_num:
                for v in range(nvec):
                    w_v[j, pl.ds(v * 16, 16)] = zero
            if with_den:
                ex_v[j, pl.ds(0, 16)] = zero
            return 0

        lax.fori_loop(0, blk_sz, zrow, 0)
        zbase = sid * rps
        for off, sz in chunked(rps):
            if do_num:
                pltpu.sync_copy(w_v.at[pl.ds(0, sz)],
                                num_sh.at[pl.ds(zbase + off, sz)])
            if with_den:
                pltpu.sync_copy(ex_v.at[pl.ds(0, sz)],
                                den_sh.at[pl.ds(zbase + off, sz)])
        plsc.subcore_barrier()

        epw = ne // _NW
        nblk = epw // blk_sz
        base0 = wid * epw

        def blk(b, _):
            base = base0 + b * blk_sz
            pltpu.sync_copy(dst_r.at[pl.ds(base, blk_sz)], idx_d)
            pltpu.sync_copy(ex_r.at[pl.ds(base, blk_sz)], ex_v)
            if do_num:
                pltpu.sync_copy(src_r.at[pl.ds(base, blk_sz)], idx_s)
                pltpu.async_copy(vt_r.at[idx_s], vt_v, sem).wait()

                def grp(g, _):
                    eids = lax.iota(i32, 16) + g * 16
                    for v in range(nvec):
                        hv = jnp.full((16,), head_lo + v, i32)
                        exs = plsc.load_gather(ex_v, [eids, hv])
                        for d in range(_DK):
                            col = jnp.full((16,), v * _DK + d, i32)
                            w = plsc.load_gather(vt_v, [eids, col]) * exs
                            plsc.store_scatter(w_v, [eids, col], w)
                    return 0

                lax.fori_loop(0, blk_sz // 16, grp, 0)
                pltpu.sync_copy(w_v, num_sh.at[idx_d], add=True)
            if with_den:
                pltpu.sync_copy(ex_v, den_sh.at[idx_d], add=True)
            return 0

        lax.fori_loop(0, nblk, blk, 0)
        plsc.subcore_barrier()

        obase = cid * npad + sid * rps
        for off, sz in chunked(rps):
            if do_num:
                pltpu.sync_copy(num_sh.at[pl.ds(zbase + off, sz)],
                                w_v.at[pl.ds(0, sz)])
                pltpu.sync_copy(w_v.at[pl.ds(0, sz)],
                                num_o.at[pl.ds(obase + off, sz)])
            if with_den:
                pltpu.sync_copy(den_sh.at[pl.ds(zbase + off, sz)],
                                ex_v.at[pl.ds(0, sz)])
                pltpu.sync_copy(ex_v.at[pl.ds(0, sz)],
                                den_o.at[pl.ds(obase + off, sz)])

    out_type = []
    scratch = []
    if do_num:
        out_type.append(jax.ShapeDtypeStruct((2 * npad, ncols), f32))
    if with_den:
        out_type.append(jax.ShapeDtypeStruct((2 * npad, _EW), f32))
    if do_num:
        scratch.append(pltpu.VMEM_SHARED((npad, ncols), f32))
    if with_den:
        scratch.append(pltpu.VMEM_SHARED((npad, _EW), f32))
    if do_num:
        scratch += [
            pltpu.VMEM((blk_sz,), i32),
            pltpu.VMEM((blk_sz, ncols), f32),
            pltpu.VMEM((blk_sz, ncols), f32),
        ]
    scratch += [
        pltpu.VMEM((blk_sz,), i32),
        pltpu.VMEM((blk_sz, _EW), f32),
        pltpu.SemaphoreType.DMA,
    ]
    res = pl.kernel(
        body,
        out_type=tuple(out_type),
        mesh=mesh,
        compiler_params=pltpu.CompilerParams(needs_layout_passes=False,
                                             use_tc_tiling_on_sc=False),
        scratch_types=scratch,
    )(vt, ex, src, dst)
    return res if isinstance(res, tuple) else (res,)


def _final(x, parts, wat, ba_row, alpha):
    n = x.shape[0]
    t = 'word' if n == _NN['word'] else ('topic' if n == _NN['topic']
                                         else 'doc')
    bn = _BN[t]
    npad = _NPAD[t]
    cblk = npad // bn
    grid = (n // bn,)

    def mkmap(cid):
        return functools.partial(
            lambda i, cid, cblk: (cid * cblk + i, 0), cid=cid, cblk=cblk)

    ins = [x]
    in_specs = [pl.BlockSpec((bn, _D), lambda i: (i, 0))]
    counts = []
    for num_arrs, den in parts:
        cnt = 0
        for arr, nc in num_arrs:
            for cid in range(2):
                ins.append(arr)
                in_specs.append(pl.BlockSpec((bn, nc), mkmap(cid)))
                cnt += 1
        for cid in range(2):
            ins.append(den)
            in_specs.append(pl.BlockSpec((bn, _EW), mkmap(cid)))
            cnt += 1
        counts.append(cnt)
    ins += [wat, ba_row, alpha]
    in_specs += [pl.BlockSpec((_D, _D), lambda i: (0, 0)),
                 pl.BlockSpec((1, _D), lambda i: (0, 0)),
                 pl.BlockSpec((1, 1), lambda i: (0, 0),
                              memory_space=pltpu.SMEM)]

    def body(x_ref, *refs):
        pos = 0
        msgs = []
        for (num_arrs, den), cnt in zip(parts, counts):
            group = refs[pos:pos + cnt]
            pos += cnt
            gi = 0
            num_cols = []
            for arr, nc in num_arrs:
                num_cols.append(group[gi][...] + group[gi + 1][...])
                gi += 2
            num = jnp.concatenate(num_cols, axis=1)
            den_v = (group[gi][...] + group[gi + 1][...])[:, :_H]
            den_rep = jnp.repeat(den_v, _DK, axis=1)
            h = num / jnp.maximum(den_rep, 1e-30)
            msgs.append(jax.nn.relu(h))
        wat_ref, ba_ref, al_ref = refs[pos], refs[pos + 1], refs[pos + 2]
        out_ref = refs[pos + 3]
        msg = msgs[0]
        for m in msgs[1:]:
            msg = msg + m
        msg = msg * (1.0 / len(msgs))
        al = al_ref[0, 0]
        trans = jnp.dot(msg, wat_ref[...],
                        preferred_element_type=jnp.float32) + ba_ref[...]
        out_ref[...] = trans * al + x_ref[...] * (1.0 - al)

    return pl.pallas_call(
        body,
        grid=grid,
        in_specs=in_specs,
        out_specs=pl.BlockSpec((bn, _D), lambda i: (i, 0)),
        out_shape=jax.ShapeDtypeStruct((n, _D), jnp.float32),
    )(*ins)


def kernel(x_word, x_topic, x_doc, src_ww, dst_ww, src_wt, dst_wt, src_wd,
           dst_wd, src_tt, dst_tt, src_td, dst_td, Wk, Wq, Wv, Wa, bk, bq,
           bv, ba, skip, rel_pri, rel_att, rel_msg):
    f32 = jnp.float32
    x = {'word': x_word, 'topic': x_topic, 'doc': x_doc}
    src = {'ww': src_ww, 'wt': src_wt, 'wd': src_wd, 'tt': src_tt,
           'td': src_td}
    dst = {'ww': dst_ww, 'wt': dst_wt, 'wd': dst_wd, 'tt': dst_tt,
           'td': dst_td}
    src = {k: v.astype(jnp.int32) for k, v in src.items()}
    dst = {k: v.astype(jnp.int32) for k, v in dst.items()}

    mk, ck, mv, cv, wqs, bqs = _combine_weights(
        Wk, Wq, Wv, bk, bq, bv, rel_att, rel_pri, rel_msg)

    wc_word = jnp.concatenate(
        [wqs[0], mk[0], mk[1], mk[2], mv[1], mv[2], mv[0]], axis=1)
    bc_word = jnp.concatenate(
        [bqs[0], ck[0], ck[1], ck[2], cv[1], cv[2], cv[0]]).reshape(1, -1)
    qs_w, kt_ww, kt_wt, kt_wd, vt_wt, vt_wd, vw0, vw1, vw2, vw3 = _prep_type(
        x['word'], wc_word, bc_word, [128, 128, 128, 128, 128, 128,
                                      32, 32, 32, 32])
    wc_topic = jnp.concatenate([wqs[1], mk[3], mk[4], mv[3], mv[4]], axis=1)
    bc_topic = jnp.concatenate(
        [bqs[1], ck[3], ck[4], cv[3], cv[4]]).reshape(1, -1)
    qs_t, kt_tt, kt_td, vt_tt, vt_td = _prep_type(
        x['topic'], wc_topic, bc_topic, [128, 128, 128, 128, 128])
    (qs_d,) = _prep_type(x['doc'], wqs[2], bqs[2].reshape(1, -1), [128])

    qs = {'word': qs_w, 'topic': qs_t, 'doc': qs_d}
    kt = {'ww': kt_ww, 'wt': kt_wt, 'wd': kt_wd, 'tt': kt_tt, 'td': kt_td}
    vt = {'wt': vt_wt, 'wd': vt_wd, 'tt': vt_tt, 'td': vt_td}
    vtww = [vw0, vw1, vw2, vw3]

    tables = [(kt[et], qs[d], src[et], dst[et], ne)
              for et, s, d, ne in _REL]
    atts = _sc_att(tables)
    att = {et: a for (et, _, _, _), a in zip(_REL, atts)}

    ex = {et: _max_exp(att[et]) for et, *_ in _REL}

    npw, npt, npd = _NPAD['word'], _NPAD['topic'], _NPAD['doc']
    ww_num = []
    for g in range(4):
        (numg,) = _sc_agg(vtww[g], ex['ww'], src['ww'], dst['ww'],
                          ne=262144, npad=npw, ncols=32, head_lo=2 * g,
                          with_den=False)
        ww_num.append(numg)
    (ww_den,) = _sc_agg(vtww[0], ex['ww'], src['ww'], dst['ww'],
                        ne=262144, npad=npw, ncols=16, head_lo=0,
                        with_den=True, do_num=False)
    num_wt, den_wt = _sc_agg(vt['wt'], ex['wt'], src['wt'], dst['wt'],
                             ne=65536, npad=npt, ncols=128, head_lo=0,
                             with_den=True)
    num_tt, den_tt = _sc_agg(vt['tt'], ex['tt'], src['tt'], dst['tt'],
                             ne=32768, npad=npt, ncols=128, head_lo=0,
                             with_den=True)
    num_wd, den_wd = _sc_agg(vt['wd'], ex['wd'], src['wd'], dst['wd'],
                             ne=131072, npad=npd, ncols=128, head_lo=0,
                             with_den=True, blk_sz=32)
    num_td, den_td = _sc_agg(vt['td'], ex['td'], src['td'], dst['td'],
                             ne=65536, npad=npd, ncols=128, head_lo=0,
                             with_den=True, blk_sz=32)

    alpha = jax.nn.sigmoid(skip).astype(f32)
    wat = jnp.transpose(Wa, (0, 2, 1))
    out_w = _final(x['word'],
                   [([(g, 32) for g in ww_num], ww_den)],
                   wat[0], ba[0].reshape(1, _D),
                   alpha[0].reshape(1, 1))
    out_t = _final(x['topic'],
                   [([(num_wt, 128)], den_wt), ([(num_tt, 128)], den_tt)],
                   wat[1], ba[1].reshape(1, _D),
                   alpha[1].reshape(1, 1))
    out_d = _final(x['doc'],
                   [([(num_wd, 128)], den_wd), ([(num_td, 128)], den_td)],
                   wat[2], ba[2].reshape(1, _D),
                   alpha[2].reshape(1, 1))
    return (out_w, out_t, out_d)


# speedup vs baseline: 1.2157x; 1.2157x over previous
import functools

import jax
import jax.numpy as jnp
import numpy as np
from jax import lax
from jax.experimental import pallas as pl
from jax.experimental.pallas import tpu as pltpu
from jax.experimental.pallas import tpu_sc as plsc

_NN = {'word': 50000, 'topic': 5000, 'doc': 10000}
_REL = [('ww', 'word', 'word', 262144), ('wt', 'word', 'topic', 65536),
        ('wd', 'word', 'doc', 131072), ('tt', 'topic', 'topic', 32768),
        ('td', 'topic', 'doc', 65536)]
_IDX = {'word': 0, 'topic': 1, 'doc': 2}
_D, _H, _DK = 128, 8, 16
_NW = 32
_B = 128
_EW = 16
_SQRT_DK = float(np.sqrt(_DK))
_NPAD = {'word': 51200, 'topic': 6400, 'doc': 12800}
_BN = {'word': 400, 'topic': 200, 'doc': 400}


def _combine_weights(Wk, Wq, Wv, bk, bq, bv, rel_att, rel_pri, rel_msg):
    def body(wk, wq, wv, bk_r, bq_r, bv_r, ratt, rpri, rmsg,
             mk_o, ck_o, mv_o, cv_o, wqs_o, bqs_o):
        for ei, (et, s, d, ne) in enumerate(_REL):
            i = _IDX[s]
            wkt = wk[i].T
            wvt = wv[i].T
            mk_cols, mv_cols = [], []
            ck_cols, cv_cols = [], []
            for h in range(_H):
                a_att = ratt[ei, h] * rpri[ei, h]
                a_msg = rmsg[ei, h]
                sl = slice(h * _DK, (h + 1) * _DK)
                mk_cols.append(wkt[:, sl] @ a_att)
                mv_cols.append(wvt[:, sl] @ a_msg)
                ck_cols.append(bk_r[i, sl].reshape(1, _DK) @ a_att)
                cv_cols.append(bv_r[i, sl].reshape(1, _DK) @ a_msg)
            mk_o[ei] = jnp.concatenate(mk_cols, axis=1)
            mv_o[ei] = jnp.concatenate(mv_cols, axis=1)
            ck_o[ei] = jnp.concatenate(ck_cols, axis=1).reshape(_D)
            cv_o[ei] = jnp.concatenate(cv_cols, axis=1).reshape(_D)
        inv = 1.0 / _SQRT_DK
        wqs_o[...] = jnp.transpose(wq[...], (0, 2, 1)) * inv
        bqs_o[...] = bq_r[...] * inv

    f32 = jnp.float32
    return pl.pallas_call(
        body,
        out_shape=(
            jax.ShapeDtypeStruct((5, _D, _D), f32),
            jax.ShapeDtypeStruct((5, _D), f32),
            jax.ShapeDtypeStruct((5, _D, _D), f32),
            jax.ShapeDtypeStruct((5, _D), f32),
            jax.ShapeDtypeStruct((3, _D, _D), f32),
            jax.ShapeDtypeStruct((3, _D), f32),
        ),
    )(Wk, Wq, Wv, bk, bq, bv, rel_att, rel_pri, rel_msg)


def _prep_type(x, wc, bc, out_cols):
    n = x.shape[0]
    bn = 1024
    grid = (pl.cdiv(n, bn),)
    ncol_total = wc.shape[1]

    def body(x_ref, wc_ref, bc_ref, *outs):
        y = jnp.dot(x_ref[...], wc_ref[...],
                    preferred_element_type=jnp.float32) + bc_ref[...]
        c0 = 0
        for o, nc in zip(outs, out_cols):
            o[...] = y[:, c0:c0 + nc]
            c0 += nc

    out_shape = tuple(jax.ShapeDtypeStruct((n, nc), jnp.float32)
                      for nc in out_cols)
    out_specs = tuple(pl.BlockSpec((bn, nc), lambda i: (i, 0))
                      for nc in out_cols)
    return pl.pallas_call(
        body,
        grid=grid,
        in_specs=[
            pl.BlockSpec((bn, _D), lambda i: (i, 0)),
            pl.BlockSpec((_D, ncol_total), lambda i: (0, 0)),
            pl.BlockSpec((1, ncol_total), lambda i: (0, 0)),
        ],
        out_specs=out_specs,
        out_shape=out_shape,
    )(x, wc, bc)


def _sc_att(tables):
    f32, i32 = jnp.float32, jnp.int32
    mesh = plsc.VectorSubcoreMesh(core_axis_name="c", subcore_axis_name="s")
    nrel = len(tables)

    def body(*refs):
        ins = refs[:4 * nrel]
        outs = refs[4 * nrel:5 * nrel]
        (idx_s0, idx_d0, kt_v0, qs_v0, att_v0,
         idx_s1, idx_d1, kt_v1, qs_v1, att_v1,
         semk0, semq0, semk1, semq1) = refs[5 * nrel:]
        bufs = ((idx_s0, idx_d0, kt_v0, qs_v0, att_v0, semk0, semq0),
                (idx_s1, idx_d1, kt_v1, qs_v1, att_v1, semk1, semq1))
        cid = lax.axis_index("c")
        sid = lax.axis_index("s")
        wid = sid * 2 + cid

        for r in range(nrel):
            kt, qs, src, dst = ins[4 * r:4 * r + 4]
            att = outs[r]
            ne = tables[r][4]
            epw = ne // _NW
            nblk = epw // _B
            base0 = wid * epw

            def issue(base, buf, kt=kt, qs=qs, src=src, dst=dst):
                idx_s, idx_d, kt_v, qs_v, _av, semk, semq = buf
                pltpu.sync_copy(src.at[pl.ds(base, _B)], idx_s)
                pltpu.sync_copy(dst.at[pl.ds(base, _B)], idx_d)
                pltpu.async_copy(kt.at[idx_s], kt_v, semk)
                pltpu.async_copy(qs.at[idx_d], qs_v, semq)

            def wait(buf, kt=kt, qs=qs):
                _is, _id, kt_v, qs_v, _av, semk, semq = buf
                pltpu.make_async_copy(kt.at[pl.ds(0, _B)], kt_v, semk).wait()
                pltpu.make_async_copy(qs.at[pl.ds(0, _B)], qs_v, semq).wait()

            def compute(base, buf, att=att):
                _is, _id, kt_v, qs_v, p_v, _sk, _sq = buf

                def edge(e, _):
                    for h in range(_H):
                        p_v[e, pl.ds(h * _DK, _DK)] = (
                            kt_v[e, pl.ds(h * _DK, _DK)] *
                            qs_v[e, pl.ds(h * _DK, _DK)])
                    return 0

                lax.fori_loop(0, _B, edge, 0, unroll=8)
                pltpu.sync_copy(p_v, att.at[pl.ds(base, _B)])

            def pair(i, _, base0=base0, nblk=nblk):
                b0 = base0 + (2 * i) * _B
                b1 = base0 + (2 * i + 1) * _B
                wait(bufs[0])
                issue(b1, bufs[1])
                compute(b0, bufs[0])
                wait(bufs[1])

                @pl.when(2 * i + 2 < nblk)
                def _():
                    issue(base0 + (2 * i + 2) * _B, bufs[0])

                compute(b1, bufs[1])
                return 0

            issue(base0, bufs[0])
            lax.fori_loop(0, nblk // 2, pair, 0)

    out_type = tuple(jax.ShapeDtypeStruct((t[4], _D), f32) for t in tables)
    args = []
    for kt, qs, src, dst, ne in tables:
        args += [kt, qs, src, dst]
    return pl.kernel(
        body,
        out_type=out_type,
        mesh=mesh,
        compiler_params=pltpu.CompilerParams(needs_layout_passes=False,
                                             use_tc_tiling_on_sc=False),
        scratch_types=[
            pltpu.VMEM((_B,), i32),
            pltpu.VMEM((_B,), i32),
            pltpu.VMEM((_B, _D), f32),
            pltpu.VMEM((_B, _D), f32),
            pltpu.VMEM((_B, _D), f32),
            pltpu.VMEM((_B,), i32),
            pltpu.VMEM((_B,), i32),
            pltpu.VMEM((_B, _D), f32),
            pltpu.VMEM((_B, _D), f32),
            pltpu.VMEM((_B, _D), f32),
            pltpu.SemaphoreType.DMA,
            pltpu.SemaphoreType.DMA,
            pltpu.SemaphoreType.DMA,
            pltpu.SemaphoreType.DMA,
        ],
    )(*args)


def _max_exp(p):
    e = p.shape[0]
    bp = 2048
    gridp = (e // bp,)

    def rm_body(p_ref, a_ref, o_ref):
        i = pl.program_id(0)
        a = jnp.sum(p_ref[...].reshape(bp, _H, _DK), axis=-1)
        a_ref[...] = a
        m = jnp.max(a, axis=0, keepdims=True)

        @pl.when(i == 0)
        def _():
            o_ref[...] = m

        @pl.when(i > 0)
        def _():
            o_ref[...] = jnp.maximum(o_ref[...], m)

    att, mx = pl.pallas_call(
        rm_body,
        grid=gridp,
        in_specs=[pl.BlockSpec((bp, _D), lambda i: (i, 0))],
        out_specs=(pl.BlockSpec((bp, _H), lambda i: (i, 0)),
                   pl.BlockSpec((1, _H), lambda i: (0, 0))),
        out_shape=(jax.ShapeDtypeStruct((e, _H), jnp.float32),
                   jax.ShapeDtypeStruct((1, _H), jnp.float32)),
    )(p)
    be = 8192
    grid = (e // be,)

    def ex_body(a_ref, m_ref, o_ref):
        ex = jnp.exp(a_ref[...] - m_ref[...])
        o_ref[...] = jnp.concatenate(
            [ex, jnp.zeros((be, _EW - _H), jnp.float32)], axis=1)

    return pl.pallas_call(
        ex_body,
        grid=grid,
        in_specs=[pl.BlockSpec((be, _H), lambda i: (i, 0)),
                  pl.BlockSpec((1, _H), lambda i: (0, 0))],
        out_specs=pl.BlockSpec((be, _EW), lambda i: (i, 0)),
        out_shape=jax.ShapeDtypeStruct((e, _EW), jnp.float32),
    )(att, mx)


def _sc_agg(vt, ex, src, dst, *, ne, npad, ncols, head_lo, with_den,
            do_num=True, blk_sz=_B):
    f32, i32 = jnp.float32, jnp.int32
    mesh = plsc.VectorSubcoreMesh(core_axis_name="c", subcore_axis_name="s")
    nvec = ncols // 16
    rps = npad // 16

    def chunked(total):
        out = []
        off = 0
        while off < total:
            sz = min(blk_sz, total - off)
            out.append((off, sz))
            off += sz
        return out

    def body(vt_r, ex_r, src_r, dst_r, *rest):
        pos = 0
        num_o = den_o = num_sh = den_sh = None
        idx_s = vt_v = w_v = None
        if do_num:
            num_o = rest[pos]
            pos += 1
        if with_den:
            den_o = rest[pos]
            pos += 1
        if do_num:
            num_sh = rest[pos]
            pos += 1
        if with_den:
            den_sh = rest[pos]
            pos += 1
        if do_num:
            idx_s, vt_v, w_v = rest[pos:pos + 3]
            pos += 3
        (idx_d, ex_v, sem) = rest[pos:]
        cid = lax.axis_index("c")
        sid = lax.axis_index("s")
        wid = sid * 2 + cid
        zero = jnp.zeros((16,), f32)

        def zrow(j, _):
            if do_num:
                for v in range(nvec):
                    w_v[j, pl.ds(v * 16, 16)] = zero
            if with_den:
                ex_v[j, pl.ds(0, 16)] = zero
            return 0

        lax.fori_loop(0, blk_sz, zrow, 0)
        zbase = sid * rps
        for off, sz in chunked(rps):
            if do_num:
                pltpu.sync_copy(w_v.at[pl.ds(0, sz)],
                                num_sh.at[pl.ds(zbase + off, sz)])
            if with_den:
                pltpu.sync_copy(ex_v.at[pl.ds(0, sz)],
                                den_sh.at[pl.ds(zbase + off, sz)])
        plsc.subcore_barrier()

        epw = ne // _NW
        nblk = epw // blk_sz
        base0 = wid * epw

        def blk(b, _):
            base = base0 + b * blk_sz
            pltpu.sync_copy(dst_r.at[pl.ds(base, blk_sz)], idx_d)
            pltpu.sync_copy(ex_r.at[pl.ds(base, blk_sz)], ex_v)
            if do_num:
                pltpu.sync_copy(src_r.at[pl.ds(base, blk_sz)], idx_s)
                pltpu.async_copy(vt_r.at[idx_s], vt_v, sem).wait()

                def grp(g, _):
                    eids = lax.iota(i32, 16) + g * 16
                    for v in range(nvec):
                        hv = jnp.full((16,), head_lo + v, i32)
                        exs = plsc.load_gather(ex_v, [eids, hv])
                        for d in range(_DK):
                            col = jnp.full((16,), v * _DK + d, i32)
                            w = plsc.load_gather(vt_v, [eids, col]) * exs
                            plsc.store_scatter(w_v, [eids, col], w)
                    return 0

                lax.fori_loop(0, blk_sz // 16, grp, 0)
                pltpu.sync_copy(w_v, num_sh.at[idx_d], add=True)
            if with_den:
                pltpu.sync_copy(ex_v, den_sh.at[idx_d], add=True)
            return 0

        lax.fori_loop(0, nblk, blk, 0)
        plsc.subcore_barrier()

        obase = cid * npad + sid * rps
        for off, sz in chunked(rps):
            if do_num:
                pltpu.sync_copy(num_sh.at[pl.ds(zbase + off, sz)],
                                w_v.at[pl.ds(0, sz)])
                pltpu.sync_copy(w_v.at[pl.ds(0, sz)],
                                num_o.at[pl.ds(obase + off, sz)])
            if with_den:
                pltpu.sync_copy(den_sh.at[pl.ds(zbase + off, sz)],
                                ex_v.at[pl.ds(0, sz)])
                pltpu.sync_copy(ex_v.at[pl.ds(0, sz)],
                                den_o.at[pl.ds(obase + off, sz)])

    out_type = []
    scratch = []
    if do_num:
        out_type.append(jax.ShapeDtypeStruct((2 * npad, ncols), f32))
    if with_den:
        out_type.append(jax.ShapeDtypeStruct((2 * npad, _EW), f32))
    if do_num:
        scratch.append(pltpu.VMEM_SHARED((npad, ncols), f32))
    if with_den:
        scratch.append(pltpu.VMEM_SHARED((npad, _EW), f32))
    if do_num:
        scratch += [
            pltpu.VMEM((blk_sz,), i32),
            pltpu.VMEM((blk_sz, ncols), f32),
            pltpu.VMEM((blk_sz, ncols), f32),
        ]
    scratch += [
        pltpu.VMEM((blk_sz,), i32),
        pltpu.VMEM((blk_sz, _EW), f32),
        pltpu.SemaphoreType.DMA,
    ]
    res = pl.kernel(
        body,
        out_type=tuple(out_type),
        mesh=mesh,
        compiler_params=pltpu.CompilerParams(needs_layout_passes=False,
                                             use_tc_tiling_on_sc=False),
        scratch_types=scratch,
    )(vt, ex, src, dst)
    return res if isinstance(res, tuple) else (res,)


def _final(x, parts, wat, ba_row, alpha):
    n = x.shape[0]
    t = 'word' if n == _NN['word'] else ('topic' if n == _NN['topic']
                                         else 'doc')
    bn = _BN[t]
    npad = _NPAD[t]
    cblk = npad // bn
    grid = (n // bn,)

    def mkmap(cid):
        return functools.partial(
            lambda i, cid, cblk: (cid * cblk + i, 0), cid=cid, cblk=cblk)

    ins = [x]
    in_specs = [pl.BlockSpec((bn, _D), lambda i: (i, 0))]
    counts = []
    for num_arrs, den in parts:
        cnt = 0
        for arr, nc in num_arrs:
            for cid in range(2):
                ins.append(arr)
                in_specs.append(pl.BlockSpec((bn, nc), mkmap(cid)))
                cnt += 1
        for cid in range(2):
            ins.append(den)
            in_specs.append(pl.BlockSpec((bn, _EW), mkmap(cid)))
            cnt += 1
        counts.append(cnt)
    ins += [wat, ba_row, alpha]
    in_specs += [pl.BlockSpec((_D, _D), lambda i: (0, 0)),
                 pl.BlockSpec((1, _D), lambda i: (0, 0)),
                 pl.BlockSpec((1, 1), lambda i: (0, 0),
                              memory_space=pltpu.SMEM)]

    def body(x_ref, *refs):
        pos = 0
        msgs = []
        for (num_arrs, den), cnt in zip(parts, counts):
            group = refs[pos:pos + cnt]
            pos += cnt
            gi = 0
            num_cols = []
            for arr, nc in num_arrs:
                num_cols.append(group[gi][...] + group[gi + 1][...])
                gi += 2
            num = jnp.concatenate(num_cols, axis=1)
            den_v = (group[gi][...] + group[gi + 1][...])[:, :_H]
            den_rep = jnp.repeat(den_v, _DK, axis=1)
            h = num / jnp.maximum(den_rep, 1e-30)
            msgs.append(jax.nn.relu(h))
        wat_ref, ba_ref, al_ref = refs[pos], refs[pos + 1], refs[pos + 2]
        out_ref = refs[pos + 3]
        msg = msgs[0]
        for m in msgs[1:]:
            msg = msg + m
        msg = msg * (1.0 / len(msgs))
        al = al_ref[0, 0]
        trans = jnp.dot(msg, wat_ref[...],
                        preferred_element_type=jnp.float32) + ba_ref[...]
        out_ref[...] = trans * al + x_ref[...] * (1.0 - al)

    return pl.pallas_call(
        body,
        grid=grid,
        in_specs=in_specs,
        out_specs=pl.BlockSpec((bn, _D), lambda i: (i, 0)),
        out_shape=jax.ShapeDtypeStruct((n, _D), jnp.float32),
    )(*ins)


def kernel(x_word, x_topic, x_doc, src_ww, dst_ww, src_wt, dst_wt, src_wd,
           dst_wd, src_tt, dst_tt, src_td, dst_td, Wk, Wq, Wv, Wa, bk, bq,
           bv, ba, skip, rel_pri, rel_att, rel_msg):
    f32 = jnp.float32
    x = {'word': x_word, 'topic': x_topic, 'doc': x_doc}
    src = {'ww': src_ww, 'wt': src_wt, 'wd': src_wd, 'tt': src_tt,
           'td': src_td}
    dst = {'ww': dst_ww, 'wt': dst_wt, 'wd': dst_wd, 'tt': dst_tt,
           'td': dst_td}
    src = {k: v.astype(jnp.int32) for k, v in src.items()}
    dst = {k: v.astype(jnp.int32) for k, v in dst.items()}

    mk, ck, mv, cv, wqs, bqs = _combine_weights(
        Wk, Wq, Wv, bk, bq, bv, rel_att, rel_pri, rel_msg)

    wc_word = jnp.concatenate(
        [wqs[0], mk[0], mk[1], mk[2], mv[1], mv[2], mv[0]], axis=1)
    bc_word = jnp.concatenate(
        [bqs[0], ck[0], ck[1], ck[2], cv[1], cv[2], cv[0]]).reshape(1, -1)
    qs_w, kt_ww, kt_wt, kt_wd, vt_wt, vt_wd, vw0, vw1, vw2, vw3 = _prep_type(
        x['word'], wc_word, bc_word, [128, 128, 128, 128, 128, 128,
                                      32, 32, 32, 32])
    wc_topic = jnp.concatenate([wqs[1], mk[3], mk[4], mv[3], mv[4]], axis=1)
    bc_topic = jnp.concatenate(
        [bqs[1], ck[3], ck[4], cv[3], cv[4]]).reshape(1, -1)
    qs_t, kt_tt, kt_td, vt_tt, vt_td = _prep_type(
        x['topic'], wc_topic, bc_topic, [128, 128, 128, 128, 128])
    (qs_d,) = _prep_type(x['doc'], wqs[2], bqs[2].reshape(1, -1), [128])

    qs = {'word': qs_w, 'topic': qs_t, 'doc': qs_d}
    kt = {'ww': kt_ww, 'wt': kt_wt, 'wd': kt_wd, 'tt': kt_tt, 'td': kt_td}
    vt = {'wt': vt_wt, 'wd': vt_wd, 'tt': vt_tt, 'td': vt_td}
    vtww = [vw0, vw1, vw2, vw3]

    tables = [(kt[et], qs[d], src[et], dst[et], ne)
              for et, s, d, ne in _REL]
    atts = _sc_att(tables)
    att = {et: a for (et, _, _, _), a in zip(_REL, atts)}

    ex = {et: _max_exp(att[et]) for et, *_ in _REL}

    npw, npt, npd = _NPAD['word'], _NPAD['topic'], _NPAD['doc']
    ww_num = []
    for g in range(4):
        (numg,) = _sc_agg(vtww[g], ex['ww'], src['ww'], dst['ww'],
                          ne=262144, npad=npw, ncols=32, head_lo=2 * g,
                          with_den=False)
        ww_num.append(numg)
    (ww_den,) = _sc_agg(vtww[0], ex['ww'], src['ww'], dst['ww'],
                        ne=262144, npad=npw, ncols=16, head_lo=0,
                        with_den=True, do_num=False)
    num_wt, den_wt = _sc_agg(vt['wt'], ex['wt'], src['wt'], dst['wt'],
                             ne=65536, npad=npt, ncols=128, head_lo=0,
                             with_den=True)
    num_tt, den_tt = _sc_agg(vt['tt'], ex['tt'], src['tt'], dst['tt'],
                             ne=32768, npad=npt, ncols=128, head_lo=0,
                             with_den=True)
    num_wd, den_wd = _sc_agg(vt['wd'], ex['wd'], src['wd'], dst['wd'],
                             ne=131072, npad=npd, ncols=128, head_lo=0,
                             with_den=True, blk_sz=32)
    num_td, den_td = _sc_agg(vt['td'], ex['td'], src['td'], dst['td'],
                             ne=65536, npad=npd, ncols=128, head_lo=0,
                             with_den=True, blk_sz=32)

    alpha = jax.nn.sigmoid(skip).astype(f32)
    wat = jnp.transpose(Wa, (0, 2, 1))
    out_w = _final(x['word'],
                   [([(g, 32) for g in ww_num], ww_den)],
                   wat[0], ba[0].reshape(1, _D),
                   alpha[0].reshape(1, 1))
    out_t = _final(x['topic'],
                   [([(num_wt, 128)], den_wt), ([(num_tt, 128)], den_tt)],
                   wat[1], ba[1].reshape(1, _D),
                   alpha[1].reshape(1, 1))
    out_d = _final(x['doc'],
                   [([(num_wd, 128)], den_wd), ([(num_td, 128)], den_td)],
                   wat[2], ba[2].reshape(1, _D),
                   alpha[2].reshape(1, 1))
    return (out_w, out_t, out_d)


# speedup vs baseline: 1.6927x; 1.3923x over previous
import functools

import jax
import jax.numpy as jnp
import numpy as np
from jax import lax
from jax.experimental import pallas as pl
from jax.experimental.pallas import tpu as pltpu
from jax.experimental.pallas import tpu_sc as plsc

_NN = {'word': 50000, 'topic': 5000, 'doc': 10000}
_REL = [('ww', 'word', 'word', 262144), ('wt', 'word', 'topic', 65536),
        ('wd', 'word', 'doc', 131072), ('tt', 'topic', 'topic', 32768),
        ('td', 'topic', 'doc', 65536)]
_IDX = {'word': 0, 'topic': 1, 'doc': 2}
_D, _H, _DK = 128, 8, 16
_NW = 32
_B = 128
_EW = 16
_SQRT_DK = float(np.sqrt(_DK))
_NPAD = {'word': 51200, 'topic': 6400, 'doc': 12800}
_BN = {'word': 400, 'topic': 200, 'doc': 400}


def _combine_weights(Wk, Wq, Wv, bk, bq, bv, rel_att, rel_pri, rel_msg):
    def body(wk, wq, wv, bk_r, bq_r, bv_r, ratt, rpri, rmsg,
             mk_o, ck_o, mv_o, cv_o, wqs_o, bqs_o):
        for ei, (et, s, d, ne) in enumerate(_REL):
            i = _IDX[s]
            wkt = wk[i].T
            wvt = wv[i].T
            mk_cols, mv_cols = [], []
            ck_cols, cv_cols = [], []
            for h in range(_H):
                a_att = ratt[ei, h] * rpri[ei, h]
                a_msg = rmsg[ei, h]
                sl = slice(h * _DK, (h + 1) * _DK)
                mk_cols.append(wkt[:, sl] @ a_att)
                mv_cols.append(wvt[:, sl] @ a_msg)
                ck_cols.append(bk_r[i, sl].reshape(1, _DK) @ a_att)
                cv_cols.append(bv_r[i, sl].reshape(1, _DK) @ a_msg)
            mk_o[ei] = jnp.concatenate(mk_cols, axis=1)
            mv_o[ei] = jnp.concatenate(mv_cols, axis=1)
            ck_o[ei] = jnp.concatenate(ck_cols, axis=1).reshape(_D)
            cv_o[ei] = jnp.concatenate(cv_cols, axis=1).reshape(_D)
        inv = 1.0 / _SQRT_DK
        wqs_o[...] = jnp.transpose(wq[...], (0, 2, 1)) * inv
        bqs_o[...] = bq_r[...] * inv

    f32 = jnp.float32
    return pl.pallas_call(
        body,
        out_shape=(
            jax.ShapeDtypeStruct((5, _D, _D), f32),
            jax.ShapeDtypeStruct((5, _D), f32),
            jax.ShapeDtypeStruct((5, _D, _D), f32),
            jax.ShapeDtypeStruct((5, _D), f32),
            jax.ShapeDtypeStruct((3, _D, _D), f32),
            jax.ShapeDtypeStruct((3, _D), f32),
        ),
    )(Wk, Wq, Wv, bk, bq, bv, rel_att, rel_pri, rel_msg)


def _prep_type(x, wc, bc, out_cols):
    n = x.shape[0]
    bn = 1024
    grid = (pl.cdiv(n, bn),)
    ncol_total = wc.shape[1]

    def body(x_ref, wc_ref, bc_ref, *outs):
        y = jnp.dot(x_ref[...], wc_ref[...],
                    preferred_element_type=jnp.float32) + bc_ref[...]
        c0 = 0
        for o, nc in zip(outs, out_cols):
            o[...] = y[:, c0:c0 + nc]
            c0 += nc

    out_shape = tuple(jax.ShapeDtypeStruct((n, nc), jnp.float32)
                      for nc in out_cols)
    out_specs = tuple(pl.BlockSpec((bn, nc), lambda i: (i, 0))
                      for nc in out_cols)
    return pl.pallas_call(
        body,
        grid=grid,
        in_specs=[
            pl.BlockSpec((bn, _D), lambda i: (i, 0)),
            pl.BlockSpec((_D, ncol_total), lambda i: (0, 0)),
            pl.BlockSpec((1, ncol_total), lambda i: (0, 0)),
        ],
        out_specs=out_specs,
        out_shape=out_shape,
    )(x, wc, bc)


def _sc_att(tables):
    f32, i32 = jnp.float32, jnp.int32
    mesh = plsc.VectorSubcoreMesh(core_axis_name="c", subcore_axis_name="s")
    nrel = len(tables)

    def body(*refs):
        ins = refs[:4 * nrel]
        outs = refs[4 * nrel:5 * nrel]
        (idx_s0, idx_d0, kt_v0, qs_v0, att_v0,
         idx_s1, idx_d1, kt_v1, qs_v1, att_v1,
         semk0, semq0, semk1, semq1) = refs[5 * nrel:]
        bufs = ((idx_s0, idx_d0, kt_v0, qs_v0, att_v0, semk0, semq0),
                (idx_s1, idx_d1, kt_v1, qs_v1, att_v1, semk1, semq1))
        cid = lax.axis_index("c")
        sid = lax.axis_index("s")
        wid = sid * 2 + cid
        lastm = lax.iota(i32, 16) == 15

        for r in range(nrel):
            kt, qs, src, dst = ins[4 * r:4 * r + 4]
            att = outs[r]
            ne = tables[r][4]
            epw = ne // _NW
            nblk = epw // _B
            base0 = wid * epw

            def issue(base, buf, kt=kt, qs=qs, src=src, dst=dst):
                idx_s, idx_d, kt_v, qs_v, _av, semk, semq = buf
                pltpu.sync_copy(src.at[pl.ds(base, _B)], idx_s)
                pltpu.sync_copy(dst.at[pl.ds(base, _B)], idx_d)
                pltpu.async_copy(kt.at[idx_s], kt_v, semk)
                pltpu.async_copy(qs.at[idx_d], qs_v, semq)

            def wait(buf, kt=kt, qs=qs):
                _is, _id, kt_v, qs_v, _av, semk, semq = buf
                pltpu.make_async_copy(kt.at[pl.ds(0, _B)], kt_v, semk).wait()
                pltpu.make_async_copy(qs.at[pl.ds(0, _B)], qs_v, semq).wait()

            def compute(base, buf, att=att):
                _is, _id, kt_v, qs_v, att_v, _sk, _sq = buf

                def edge(e, _):
                    ev = jnp.full((16,), e, i32)
                    for h in range(_H):
                        p = (kt_v[e, pl.ds(h * _DK, _DK)] *
                             qs_v[e, pl.ds(h * _DK, _DK)])
                        c = plsc.cumsum(p)
                        plsc.store_scatter(
                            att_v, [ev, jnp.full((16,), h, i32)], c,
                            mask=lastm)
                    return 0

                lax.fori_loop(0, _B, edge, 0, unroll=8)
                pltpu.sync_copy(att_v, att.at[pl.ds(base, _B)])

            def pair(i, _, base0=base0, nblk=nblk):
                b0 = base0 + (2 * i) * _B
                b1 = base0 + (2 * i + 1) * _B
                wait(bufs[0])
                issue(b1, bufs[1])
                compute(b0, bufs[0])
                wait(bufs[1])

                @pl.when(2 * i + 2 < nblk)
                def _():
                    issue(base0 + (2 * i + 2) * _B, bufs[0])

                compute(b1, bufs[1])
                return 0

            issue(base0, bufs[0])
            lax.fori_loop(0, nblk // 2, pair, 0)

    out_type = tuple(jax.ShapeDtypeStruct((t[4], _H), f32) for t in tables)
    args = []
    for kt, qs, src, dst, ne in tables:
        args += [kt, qs, src, dst]
    return pl.kernel(
        body,
        out_type=out_type,
        mesh=mesh,
        compiler_params=pltpu.CompilerParams(needs_layout_passes=False,
                                             use_tc_tiling_on_sc=False),
        scratch_types=[
            pltpu.VMEM((_B,), i32),
            pltpu.VMEM((_B,), i32),
            pltpu.VMEM((_B, _D), f32),
            pltpu.VMEM((_B, _D), f32),
            pltpu.VMEM((_B, _H), f32),
            pltpu.VMEM((_B,), i32),
            pltpu.VMEM((_B,), i32),
            pltpu.VMEM((_B, _D), f32),
            pltpu.VMEM((_B, _D), f32),
            pltpu.VMEM((_B, _H), f32),
            pltpu.SemaphoreType.DMA,
            pltpu.SemaphoreType.DMA,
            pltpu.SemaphoreType.DMA,
            pltpu.SemaphoreType.DMA,
        ],
    )(*args)


def _max_exp(att):
    e = att.shape[0]
    be = 8192
    grid = (e // be,)

    def mx_body(a_ref, o_ref):
        i = pl.program_id(0)
        m = jnp.max(a_ref[...], axis=0, keepdims=True)

        @pl.when(i == 0)
        def _():
            o_ref[...] = m

        @pl.when(i > 0)
        def _():
            o_ref[...] = jnp.maximum(o_ref[...], m)

    mx = pl.pallas_call(
        mx_body,
        grid=grid,
        in_specs=[pl.BlockSpec((be, _H), lambda i: (i, 0))],
        out_specs=pl.BlockSpec((1, _H), lambda i: (0, 0)),
        out_shape=jax.ShapeDtypeStruct((1, _H), jnp.float32),
    )(att)

    def ex_body(a_ref, m_ref, o_ref):
        ex = jnp.exp(a_ref[...] - m_ref[...])
        o_ref[...] = jnp.concatenate(
            [ex, jnp.zeros((be, _EW - _H), jnp.float32)], axis=1)

    return pl.pallas_call(
        ex_body,
        grid=grid,
        in_specs=[pl.BlockSpec((be, _H), lambda i: (i, 0)),
                  pl.BlockSpec((1, _H), lambda i: (0, 0))],
        out_specs=pl.BlockSpec((be, _EW), lambda i: (i, 0)),
        out_shape=jax.ShapeDtypeStruct((e, _EW), jnp.float32),
    )(att, mx)


def _sc_agg(vt, ex, src, dst, *, ne, npad, ncols, head_lo, with_den,
            do_num=True, blk_sz=_B):
    f32, i32 = jnp.float32, jnp.int32
    mesh = plsc.VectorSubcoreMesh(core_axis_name="c", subcore_axis_name="s")
    nvec = ncols // 16
    rps = npad // 16

    def chunked(total):
        out = []
        off = 0
        while off < total:
            sz = min(blk_sz, total - off)
            out.append((off, sz))
            off += sz
        return out

    def body(vt_r, ex_r, src_r, dst_r, *rest):
        pos = 0
        num_o = den_o = num_sh = den_sh = None
        idx_s = vt_v = w_v = None
        if do_num:
            num_o = rest[pos]
            pos += 1
        if with_den:
            den_o = rest[pos]
            pos += 1
        if do_num:
            num_sh = rest[pos]
            pos += 1
        if with_den:
            den_sh = rest[pos]
            pos += 1
        if do_num:
            idx_s0, vt_v0, idx_s1, vt_v1, w_v = rest[pos:pos + 5]
            pos += 5
        else:
            idx_s0 = vt_v0 = idx_s1 = vt_v1 = w_v = None
        (idx_d0, ex_v0, idx_d1, ex_v1, sem0, sem1) = rest[pos:]
        bufs2 = ((idx_s0, idx_d0, vt_v0, ex_v0, sem0),
                 (idx_s1, idx_d1, vt_v1, ex_v1, sem1))
        idx_d, ex_v = idx_d0, ex_v0
        cid = lax.axis_index("c")
        sid = lax.axis_index("s")
        wid = sid * 2 + cid
        zero = jnp.zeros((16,), f32)

        def zrow(j, _):
            if do_num:
                for v in range(nvec):
                    w_v[j, pl.ds(v * 16, 16)] = zero
            if with_den:
                ex_v[j, pl.ds(0, 16)] = zero
            return 0

        lax.fori_loop(0, blk_sz, zrow, 0)
        zbase = sid * rps
        for off, sz in chunked(rps):
            if do_num:
                pltpu.sync_copy(w_v.at[pl.ds(0, sz)],
                                num_sh.at[pl.ds(zbase + off, sz)])
            if with_den:
                pltpu.sync_copy(ex_v.at[pl.ds(0, sz)],
                                den_sh.at[pl.ds(zbase + off, sz)])
        plsc.subcore_barrier()

        epw = ne // _NW
        nblk = epw // blk_sz
        base0 = wid * epw

        def issue(base, buf):
            b_idx_s, b_idx_d, b_vt_v, b_ex_v, b_sem = buf
            pltpu.sync_copy(dst_r.at[pl.ds(base, blk_sz)], b_idx_d)
            pltpu.sync_copy(ex_r.at[pl.ds(base, blk_sz)], b_ex_v)
            if do_num:
                pltpu.sync_copy(src_r.at[pl.ds(base, blk_sz)], b_idx_s)
                pltpu.async_copy(vt_r.at[b_idx_s], b_vt_v, b_sem)

        def waitg(buf):
            if do_num:
                _i, _j, b_vt_v, _e, b_sem = buf
                pltpu.make_async_copy(vt_r.at[pl.ds(0, blk_sz)],
                                      b_vt_v, b_sem).wait()

        def process(buf):
            _i, b_idx_d, b_vt_v, b_ex_v, _s = buf
            if do_num:
                def edge(e, _):
                    ev = jnp.full((16,), e, i32)
                    for v in range(nvec):
                        hv = jnp.full((16,), head_lo + v, i32)
                        exs = plsc.load_gather(b_ex_v, [ev, hv])
                        w_v[e, pl.ds(v * 16, 16)] = (
                            b_vt_v[e, pl.ds(v * 16, 16)] * exs)
                    return 0

                lax.fori_loop(0, blk_sz, edge, 0, unroll=8)
                pltpu.sync_copy(w_v, num_sh.at[b_idx_d], add=True)
            if with_den:
                pltpu.sync_copy(b_ex_v, den_sh.at[b_idx_d], add=True)

        def pair(i, _):
            b0 = base0 + (2 * i) * blk_sz
            b1 = base0 + (2 * i + 1) * blk_sz
            waitg(bufs2[0])
            issue(b1, bufs2[1])
            process(bufs2[0])
            waitg(bufs2[1])

            @pl.when(2 * i + 2 < nblk)
            def _():
                issue(base0 + (2 * i + 2) * blk_sz, bufs2[0])

            process(bufs2[1])
            return 0

        issue(base0, bufs2[0])
        lax.fori_loop(0, nblk // 2, pair, 0)
        plsc.subcore_barrier()

        obase = cid * npad + sid * rps
        for off, sz in chunked(rps):
            if do_num:
                pltpu.sync_copy(num_sh.at[pl.ds(zbase + off, sz)],
                                w_v.at[pl.ds(0, sz)])
                pltpu.sync_copy(w_v.at[pl.ds(0, sz)],
                                num_o.at[pl.ds(obase + off, sz)])
            if with_den:
                pltpu.sync_copy(den_sh.at[pl.ds(zbase + off, sz)],
                                ex_v.at[pl.ds(0, sz)])
                pltpu.sync_copy(ex_v.at[pl.ds(0, sz)],
                                den_o.at[pl.ds(obase + off, sz)])

    out_type = []
    scratch = []
    if do_num:
        out_type.append(jax.ShapeDtypeStruct((2 * npad, ncols), f32))
    if with_den:
        out_type.append(jax.ShapeDtypeStruct((2 * npad, _EW), f32))
    if do_num:
        scratch.append(pltpu.VMEM_SHARED((npad, ncols), f32))
    if with_den:
        scratch.append(pltpu.VMEM_SHARED((npad, _EW), f32))
    if do_num:
        scratch += [
            pltpu.VMEM((blk_sz,), i32),
            pltpu.VMEM((blk_sz, ncols), f32),
            pltpu.VMEM((blk_sz,), i32),
            pltpu.VMEM((blk_sz, ncols), f32),
            pltpu.VMEM((blk_sz, ncols), f32),
        ]
    scratch += [
        pltpu.VMEM((blk_sz,), i32),
        pltpu.VMEM((blk_sz, _EW), f32),
        pltpu.VMEM((blk_sz,), i32),
        pltpu.VMEM((blk_sz, _EW), f32),
        pltpu.SemaphoreType.DMA,
        pltpu.SemaphoreType.DMA,
    ]
    res = pl.kernel(
        body,
        out_type=tuple(out_type),
        mesh=mesh,
        compiler_params=pltpu.CompilerParams(needs_layout_passes=False,
                                             use_tc_tiling_on_sc=False),
        scratch_types=scratch,
    )(vt, ex, src, dst)
    return res if isinstance(res, tuple) else (res,)


def _final(x, parts, wat, ba_row, alpha):
    n = x.shape[0]
    t = 'word' if n == _NN['word'] else ('topic' if n == _NN['topic']
                                         else 'doc')
    bn = _BN[t]
    npad = _NPAD[t]
    cblk = npad // bn
    grid = (n // bn,)

    def mkmap(cid):
        return functools.partial(
            lambda i, cid, cblk: (cid * cblk + i, 0), cid=cid, cblk=cblk)

    ins = [x]
    in_specs = [pl.BlockSpec((bn, _D), lambda i: (i, 0))]
    counts = []
    for num_arrs, den in parts:
        cnt = 0
        for arr, nc in num_arrs:
            for cid in range(2):
                ins.append(arr)
                in_specs.append(pl.BlockSpec((bn, nc), mkmap(cid)))
                cnt += 1
        for cid in range(2):
            ins.append(den)
            in_specs.append(pl.BlockSpec((bn, _EW), mkmap(cid)))
            cnt += 1
        counts.append(cnt)
    ins += [wat, ba_row, alpha]
    in_specs += [pl.BlockSpec((_D, _D), lambda i: (0, 0)),
                 pl.BlockSpec((1, _D), lambda i: (0, 0)),
                 pl.BlockSpec((1, 1), lambda i: (0, 0),
                              memory_space=pltpu.SMEM)]

    def body(x_ref, *refs):
        pos = 0
        msgs = []
        for (num_arrs, den), cnt in zip(parts, counts):
            group = refs[pos:pos + cnt]
            pos += cnt
            gi = 0
            num_cols = []
            for arr, nc in num_arrs:
                num_cols.append(group[gi][...] + group[gi + 1][...])
                gi += 2
            num = jnp.concatenate(num_cols, axis=1)
            den_v = (group[gi][...] + group[gi + 1][...])[:, :_H]
            den_rep = jnp.repeat(den_v, _DK, axis=1)
            h = num / jnp.maximum(den_rep, 1e-30)
            msgs.append(jax.nn.relu(h))
        wat_ref, ba_ref, al_ref = refs[pos], refs[pos + 1], refs[pos + 2]
        out_ref = refs[pos + 3]
        msg = msgs[0]
        for m in msgs[1:]:
            msg = msg + m
        msg = msg * (1.0 / len(msgs))
        al = al_ref[0, 0]
        trans = jnp.dot(msg, wat_ref[...],
                        preferred_element_type=jnp.float32) + ba_ref[...]
        out_ref[...] = trans * al + x_ref[...] * (1.0 - al)

    return pl.pallas_call(
        body,
        grid=grid,
        in_specs=in_specs,
        out_specs=pl.BlockSpec((bn, _D), lambda i: (i, 0)),
        out_shape=jax.ShapeDtypeStruct((n, _D), jnp.float32),
    )(*ins)


def kernel(x_word, x_topic, x_doc, src_ww, dst_ww, src_wt, dst_wt, src_wd,
           dst_wd, src_tt, dst_tt, src_td, dst_td, Wk, Wq, Wv, Wa, bk, bq,
           bv, ba, skip, rel_pri, rel_att, rel_msg):
    f32 = jnp.float32
    x = {'word': x_word, 'topic': x_topic, 'doc': x_doc}
    src = {'ww': src_ww, 'wt': src_wt, 'wd': src_wd, 'tt': src_tt,
           'td': src_td}
    dst = {'ww': dst_ww, 'wt': dst_wt, 'wd': dst_wd, 'tt': dst_tt,
           'td': dst_td}
    src = {k: v.astype(jnp.int32) for k, v in src.items()}
    dst = {k: v.astype(jnp.int32) for k, v in dst.items()}

    mk, ck, mv, cv, wqs, bqs = _combine_weights(
        Wk, Wq, Wv, bk, bq, bv, rel_att, rel_pri, rel_msg)

    wc_word = jnp.concatenate(
        [wqs[0], mk[0], mk[1], mk[2], mv[1], mv[2], mv[0]], axis=1)
    bc_word = jnp.concatenate(
        [bqs[0], ck[0], ck[1], ck[2], cv[1], cv[2], cv[0]]).reshape(1, -1)
    qs_w, kt_ww, kt_wt, kt_wd, vt_wt, vt_wd, vw0, vw1, vw2, vw3 = _prep_type(
        x['word'], wc_word, bc_word, [128, 128, 128, 128, 128, 128,
                                      32, 32, 32, 32])
    wc_topic = jnp.concatenate([wqs[1], mk[3], mk[4], mv[3], mv[4]], axis=1)
    bc_topic = jnp.concatenate(
        [bqs[1], ck[3], ck[4], cv[3], cv[4]]).reshape(1, -1)
    qs_t, kt_tt, kt_td, vt_tt, vt_td = _prep_type(
        x['topic'], wc_topic, bc_topic, [128, 128, 128, 128, 128])
    (qs_d,) = _prep_type(x['doc'], wqs[2], bqs[2].reshape(1, -1), [128])

    qs = {'word': qs_w, 'topic': qs_t, 'doc': qs_d}
    kt = {'ww': kt_ww, 'wt': kt_wt, 'wd': kt_wd, 'tt': kt_tt, 'td': kt_td}
    vt = {'wt': vt_wt, 'wd': vt_wd, 'tt': vt_tt, 'td': vt_td}
    vtww = [vw0, vw1, vw2, vw3]

    tables = [(kt[et], qs[d], src[et], dst[et], ne)
              for et, s, d, ne in _REL]
    atts = _sc_att(tables)
    att = {et: a for (et, _, _, _), a in zip(_REL, atts)}

    ex = {et: _max_exp(att[et]) for et, *_ in _REL}

    npw, npt, npd = _NPAD['word'], _NPAD['topic'], _NPAD['doc']
    ww_num = []
    for g in range(4):
        (numg,) = _sc_agg(vtww[g], ex['ww'], src['ww'], dst['ww'],
                          ne=262144, npad=npw, ncols=32, head_lo=2 * g,
                          with_den=False)
        ww_num.append(numg)
    (ww_den,) = _sc_agg(vtww[0], ex['ww'], src['ww'], dst['ww'],
                        ne=262144, npad=npw, ncols=16, head_lo=0,
                        with_den=True, do_num=False)
    num_wt, den_wt = _sc_agg(vt['wt'], ex['wt'], src['wt'], dst['wt'],
                             ne=65536, npad=npt, ncols=128, head_lo=0,
                             with_den=True)
    num_tt, den_tt = _sc_agg(vt['tt'], ex['tt'], src['tt'], dst['tt'],
                             ne=32768, npad=npt, ncols=128, head_lo=0,
                             with_den=True)
    num_wd, den_wd = _sc_agg(vt['wd'], ex['wd'], src['wd'], dst['wd'],
                             ne=131072, npad=npd, ncols=128, head_lo=0,
                             with_den=True, blk_sz=32)
    num_td, den_td = _sc_agg(vt['td'], ex['td'], src['td'], dst['td'],
                             ne=65536, npad=npd, ncols=128, head_lo=0,
                             with_den=True, blk_sz=32)

    alpha = jax.nn.sigmoid(skip).astype(f32)
    wat = jnp.transpose(Wa, (0, 2, 1))
    out_w = _final(x['word'],
                   [([(g, 32) for g in ww_num], ww_den)],
                   wat[0], ba[0].reshape(1, _D),
                   alpha[0].reshape(1, 1))
    out_t = _final(x['topic'],
                   [([(num_wt, 128)], den_wt), ([(num_tt, 128)], den_tt)],
                   wat[1], ba[1].reshape(1, _D),
                   alpha[1].reshape(1, 1))
    out_d = _final(x['doc'],
                   [([(num_wd, 128)], den_wd), ([(num_td, 128)], den_td)],
                   wat[2], ba[2].reshape(1, _D),
                   alpha[2].reshape(1, 1))
    return (out_w, out_t, out_d)


# speedup vs baseline: 1.7441x; 1.0303x over previous
import functools

import jax
import jax.numpy as jnp
import numpy as np
from jax import lax
from jax.experimental import pallas as pl
from jax.experimental.pallas import tpu as pltpu
from jax.experimental.pallas import tpu_sc as plsc

_NN = {'word': 50000, 'topic': 5000, 'doc': 10000}
_REL = [('ww', 'word', 'word', 262144), ('wt', 'word', 'topic', 65536),
        ('wd', 'word', 'doc', 131072), ('tt', 'topic', 'topic', 32768),
        ('td', 'topic', 'doc', 65536)]
_IDX = {'word': 0, 'topic': 1, 'doc': 2}
_D, _H, _DK = 128, 8, 16
_NW = 32
_B = 128
_EW = 16
_SQRT_DK = float(np.sqrt(_DK))
_NPAD = {'word': 51200, 'topic': 6400, 'doc': 12800}
_BN = {'word': 400, 'topic': 200, 'doc': 400}


def _combine_weights(Wk, Wq, Wv, bk, bq, bv, rel_att, rel_pri, rel_msg):
    def body(wk, wq, wv, bk_r, bq_r, bv_r, ratt, rpri, rmsg,
             mk_o, ck_o, mv_o, cv_o, wqs_o, bqs_o):
        for ei, (et, s, d, ne) in enumerate(_REL):
            i = _IDX[s]
            wkt = wk[i].T
            wvt = wv[i].T
            mk_cols, mv_cols = [], []
            ck_cols, cv_cols = [], []
            for h in range(_H):
                a_att = ratt[ei, h] * rpri[ei, h]
                a_msg = rmsg[ei, h]
                sl = slice(h * _DK, (h + 1) * _DK)
                mk_cols.append(wkt[:, sl] @ a_att)
                mv_cols.append(wvt[:, sl] @ a_msg)
                ck_cols.append(bk_r[i, sl].reshape(1, _DK) @ a_att)
                cv_cols.append(bv_r[i, sl].reshape(1, _DK) @ a_msg)
            mk_o[ei] = jnp.concatenate(mk_cols, axis=1)
            mv_o[ei] = jnp.concatenate(mv_cols, axis=1)
            ck_o[ei] = jnp.concatenate(ck_cols, axis=1).reshape(_D)
            cv_o[ei] = jnp.concatenate(cv_cols, axis=1).reshape(_D)
        inv = 1.0 / _SQRT_DK
        wqs_o[...] = jnp.transpose(wq[...], (0, 2, 1)) * inv
        bqs_o[...] = bq_r[...] * inv

    f32 = jnp.float32
    return pl.pallas_call(
        body,
        out_shape=(
            jax.ShapeDtypeStruct((5, _D, _D), f32),
            jax.ShapeDtypeStruct((5, _D), f32),
            jax.ShapeDtypeStruct((5, _D, _D), f32),
            jax.ShapeDtypeStruct((5, _D), f32),
            jax.ShapeDtypeStruct((3, _D, _D), f32),
            jax.ShapeDtypeStruct((3, _D), f32),
        ),
    )(Wk, Wq, Wv, bk, bq, bv, rel_att, rel_pri, rel_msg)


def _prep_type(x, wc, bc, out_cols):
    n = x.shape[0]
    bn = 1024
    grid = (pl.cdiv(n, bn),)
    ncol_total = wc.shape[1]

    def body(x_ref, wc_ref, bc_ref, *outs):
        y = jnp.dot(x_ref[...], wc_ref[...],
                    preferred_element_type=jnp.float32) + bc_ref[...]
        c0 = 0
        for o, nc in zip(outs, out_cols):
            o[...] = y[:, c0:c0 + nc]
            c0 += nc

    out_shape = tuple(jax.ShapeDtypeStruct((n, nc), jnp.float32)
                      for nc in out_cols)
    out_specs = tuple(pl.BlockSpec((bn, nc), lambda i: (i, 0))
                      for nc in out_cols)
    return pl.pallas_call(
        body,
        grid=grid,
        in_specs=[
            pl.BlockSpec((bn, _D), lambda i: (i, 0)),
            pl.BlockSpec((_D, ncol_total), lambda i: (0, 0)),
            pl.BlockSpec((1, ncol_total), lambda i: (0, 0)),
        ],
        out_specs=out_specs,
        out_shape=out_shape,
    )(x, wc, bc)


def _sc_att(tables):
    f32, i32 = jnp.float32, jnp.int32
    mesh = plsc.VectorSubcoreMesh(core_axis_name="c", subcore_axis_name="s")
    nrel = len(tables)

    def body(*refs):
        ins = refs[:4 * nrel]
        outs = refs[4 * nrel:5 * nrel]
        (srcall, dstall,
         idx_s0, idx_d0, kt_v0, qs_v0, att_v0,
         idx_s1, idx_d1, kt_v1, qs_v1, att_v1,
         semk0, semq0, semk1, semq1) = refs[5 * nrel:]
        bufs = ((idx_s0, idx_d0, kt_v0, qs_v0, att_v0, semk0, semq0),
                (idx_s1, idx_d1, kt_v1, qs_v1, att_v1, semk1, semq1))
        cid = lax.axis_index("c")
        sid = lax.axis_index("s")
        wid = sid * 2 + cid
        lastm = lax.iota(i32, 16) == 15

        for r in range(nrel):
            kt, qs, src, dst = ins[4 * r:4 * r + 4]
            att = outs[r]
            ne = tables[r][4]
            epw = ne // _NW
            nblk = epw // _B
            base0 = wid * epw

            def issue(loff, buf, kt=kt, qs=qs):
                _is, _id, kt_v, qs_v, _av, semk, semq = buf
                pltpu.async_copy(kt.at[srcall.at[pl.ds(loff, _B)]],
                                 kt_v, semk)
                pltpu.async_copy(qs.at[dstall.at[pl.ds(loff, _B)]],
                                 qs_v, semq)

            def wait(buf, kt=kt, qs=qs):
                _is, _id, kt_v, qs_v, _av, semk, semq = buf
                pltpu.make_async_copy(kt.at[pl.ds(0, _B)], kt_v, semk).wait()
                pltpu.make_async_copy(qs.at[pl.ds(0, _B)], qs_v, semq).wait()

            def compute(base, buf, att=att):
                _is, _id, kt_v, qs_v, att_v, _sk, _sq = buf

                def edge(e, _):
                    ev = jnp.full((16,), e, i32)
                    for h in range(_H):
                        p = (kt_v[e, pl.ds(h * _DK, _DK)] *
                             qs_v[e, pl.ds(h * _DK, _DK)])
                        c = plsc.cumsum(p)
                        plsc.store_scatter(
                            att_v, [ev, jnp.full((16,), h, i32)], c,
                            mask=lastm)
                    return 0

                lax.fori_loop(0, _B, edge, 0, unroll=8)
                pltpu.sync_copy(att_v, att.at[pl.ds(base, _B)])

            def pair(i, _, base0=base0, nblk=nblk):
                b0 = base0 + (2 * i) * _B
                b1 = base0 + (2 * i + 1) * _B
                wait(bufs[0])
                issue((2 * i + 1) * _B, bufs[1])
                compute(b0, bufs[0])
                wait(bufs[1])

                @pl.when(2 * i + 2 < nblk)
                def _():
                    issue((2 * i + 2) * _B, bufs[0])

                compute(b1, bufs[1])
                return 0

            pltpu.sync_copy(src.at[pl.ds(base0, epw)],
                            srcall.at[pl.ds(0, epw)])
            pltpu.sync_copy(dst.at[pl.ds(base0, epw)],
                            dstall.at[pl.ds(0, epw)])
            issue(0, bufs[0])
            lax.fori_loop(0, nblk // 2, pair, 0)

    out_type = tuple(jax.ShapeDtypeStruct((t[4], _H), f32) for t in tables)
    args = []
    for kt, qs, src, dst, ne in tables:
        args += [kt, qs, src, dst]
    return pl.kernel(
        body,
        out_type=out_type,
        mesh=mesh,
        compiler_params=pltpu.CompilerParams(needs_layout_passes=False,
                                             use_tc_tiling_on_sc=False),
        scratch_types=[
            pltpu.VMEM((8192,), i32),
            pltpu.VMEM((8192,), i32),
            pltpu.VMEM((_B,), i32),
            pltpu.VMEM((_B,), i32),
            pltpu.VMEM((_B, _D), f32),
            pltpu.VMEM((_B, _D), f32),
            pltpu.VMEM((_B, _H), f32),
            pltpu.VMEM((_B,), i32),
            pltpu.VMEM((_B,), i32),
            pltpu.VMEM((_B, _D), f32),
            pltpu.VMEM((_B, _D), f32),
            pltpu.VMEM((_B, _H), f32),
            pltpu.SemaphoreType.DMA,
            pltpu.SemaphoreType.DMA,
            pltpu.SemaphoreType.DMA,
            pltpu.SemaphoreType.DMA,
        ],
    )(*args)


def _max_exp(att):
    e = att.shape[0]
    be = 8192
    grid = (e // be,)

    def mx_body(a_ref, o_ref):
        i = pl.program_id(0)
        m = jnp.max(a_ref[...], axis=0, keepdims=True)

        @pl.when(i == 0)
        def _():
            o_ref[...] = m

        @pl.when(i > 0)
        def _():
            o_ref[...] = jnp.maximum(o_ref[...], m)

    mx = pl.pallas_call(
        mx_body,
        grid=grid,
        in_specs=[pl.BlockSpec((be, _H), lambda i: (i, 0))],
        out_specs=pl.BlockSpec((1, _H), lambda i: (0, 0)),
        out_shape=jax.ShapeDtypeStruct((1, _H), jnp.float32),
    )(att)

    def ex_body(a_ref, m_ref, o_ref):
        ex = jnp.exp(a_ref[...] - m_ref[...])
        o_ref[...] = jnp.concatenate(
            [ex, jnp.zeros((be, _EW - _H), jnp.float32)], axis=1)

    return pl.pallas_call(
        ex_body,
        grid=grid,
        in_specs=[pl.BlockSpec((be, _H), lambda i: (i, 0)),
                  pl.BlockSpec((1, _H), lambda i: (0, 0))],
        out_specs=pl.BlockSpec((be, _EW), lambda i: (i, 0)),
        out_shape=jax.ShapeDtypeStruct((e, _EW), jnp.float32),
    )(att, mx)


def _sc_agg(vt, ex, src, dst, *, ne, npad, ncols, head_lo, with_den,
            do_num=True, blk_sz=_B):
    f32, i32 = jnp.float32, jnp.int32
    mesh = plsc.VectorSubcoreMesh(core_axis_name="c", subcore_axis_name="s")
    nvec = ncols // 16
    rps = npad // 16

    def chunked(total):
        out = []
        off = 0
        while off < total:
            sz = min(blk_sz, total - off)
            out.append((off, sz))
            off += sz
        return out

    def body(vt_r, ex_r, src_r, dst_r, *rest):
        pos = 0
        num_o = den_o = num_sh = den_sh = None
        idx_s = vt_v = w_v = None
        if do_num:
            num_o = rest[pos]
            pos += 1
        if with_den:
            den_o = rest[pos]
            pos += 1
        if do_num:
            num_sh = rest[pos]
            pos += 1
        if with_den:
            den_sh = rest[pos]
            pos += 1
        if do_num:
            idx_s0, vt_v0, idx_s1, vt_v1, w_v = rest[pos:pos + 5]
            pos += 5
        else:
            idx_s0 = vt_v0 = idx_s1 = vt_v1 = w_v = None
        (idx_d0, ex_v0, idx_d1, ex_v1, sem0, sem1) = rest[pos:]
        bufs2 = ((idx_s0, idx_d0, vt_v0, ex_v0, sem0),
                 (idx_s1, idx_d1, vt_v1, ex_v1, sem1))
        idx_d, ex_v = idx_d0, ex_v0
        cid = lax.axis_index("c")
        sid = lax.axis_index("s")
        wid = sid * 2 + cid
        zero = jnp.zeros((16,), f32)

        def zrow(j, _):
            if do_num:
                for v in range(nvec):
                    w_v[j, pl.ds(v * 16, 16)] = zero
            if with_den:
                ex_v[j, pl.ds(0, 16)] = zero
            return 0

        lax.fori_loop(0, blk_sz, zrow, 0)
        zbase = sid * rps
        for off, sz in chunked(rps):
            if do_num:
                pltpu.sync_copy(w_v.at[pl.ds(0, sz)],
                                num_sh.at[pl.ds(zbase + off, sz)])
            if with_den:
                pltpu.sync_copy(ex_v.at[pl.ds(0, sz)],
                                den_sh.at[pl.ds(zbase + off, sz)])
        plsc.subcore_barrier()

        epw = ne // _NW
        nblk = epw // blk_sz
        base0 = wid * epw

        def issue(base, buf):
            b_idx_s, b_idx_d, b_vt_v, b_ex_v, b_sem = buf
            pltpu.sync_copy(dst_r.at[pl.ds(base, blk_sz)], b_idx_d)
            pltpu.sync_copy(ex_r.at[pl.ds(base, blk_sz)], b_ex_v)
            if do_num:
                pltpu.sync_copy(src_r.at[pl.ds(base, blk_sz)], b_idx_s)
                pltpu.async_copy(vt_r.at[b_idx_s], b_vt_v, b_sem)

        def waitg(buf):
            if do_num:
                _i, _j, b_vt_v, _e, b_sem = buf
                pltpu.make_async_copy(vt_r.at[pl.ds(0, blk_sz)],
                                      b_vt_v, b_sem).wait()

        def process(buf):
            _i, b_idx_d, b_vt_v, b_ex_v, _s = buf
            if do_num:
                def edge(e, _):
                    ev = jnp.full((16,), e, i32)
                    for v in range(nvec):
                        hv = jnp.full((16,), head_lo + v, i32)
                        exs = plsc.load_gather(b_ex_v, [ev, hv])
                        w_v[e, pl.ds(v * 16, 16)] = (
                            b_vt_v[e, pl.ds(v * 16, 16)] * exs)
                    return 0

                lax.fori_loop(0, blk_sz, edge, 0, unroll=8)
                pltpu.sync_copy(w_v, num_sh.at[b_idx_d], add=True)
            if with_den:
                pltpu.sync_copy(b_ex_v, den_sh.at[b_idx_d], add=True)

        def pair(i, _):
            b0 = base0 + (2 * i) * blk_sz
            b1 = base0 + (2 * i + 1) * blk_sz
            waitg(bufs2[0])
            issue(b1, bufs2[1])
            process(bufs2[0])
            waitg(bufs2[1])

            @pl.when(2 * i + 2 < nblk)
            def _():
                issue(base0 + (2 * i + 2) * blk_sz, bufs2[0])

            process(bufs2[1])
            return 0

        issue(base0, bufs2[0])
        lax.fori_loop(0, nblk // 2, pair, 0)
        plsc.subcore_barrier()

        obase = cid * npad + sid * rps
        for off, sz in chunked(rps):
            if do_num:
                pltpu.sync_copy(num_sh.at[pl.ds(zbase + off, sz)],
                                w_v.at[pl.ds(0, sz)])
                pltpu.sync_copy(w_v.at[pl.ds(0, sz)],
                                num_o.at[pl.ds(obase + off, sz)])
            if with_den:
                pltpu.sync_copy(den_sh.at[pl.ds(zbase + off, sz)],
                                ex_v.at[pl.ds(0, sz)])
                pltpu.sync_copy(ex_v.at[pl.ds(0, sz)],
                                den_o.at[pl.ds(obase + off, sz)])

    out_type = []
    scratch = []
    if do_num:
        out_type.append(jax.ShapeDtypeStruct((2 * npad, ncols), f32))
    if with_den:
        out_type.append(jax.ShapeDtypeStruct((2 * npad, _EW), f32))
    if do_num:
        scratch.append(pltpu.VMEM_SHARED((npad, ncols), f32))
    if with_den:
        scratch.append(pltpu.VMEM_SHARED((npad, _EW), f32))
    if do_num:
        scratch += [
            pltpu.VMEM((blk_sz,), i32),
            pltpu.VMEM((blk_sz, ncols), f32),
            pltpu.VMEM((blk_sz,), i32),
            pltpu.VMEM((blk_sz, ncols), f32),
            pltpu.VMEM((blk_sz, ncols), f32),
        ]
    scratch += [
        pltpu.VMEM((blk_sz,), i32),
        pltpu.VMEM((blk_sz, _EW), f32),
        pltpu.VMEM((blk_sz,), i32),
        pltpu.VMEM((blk_sz, _EW), f32),
        pltpu.SemaphoreType.DMA,
        pltpu.SemaphoreType.DMA,
    ]
    res = pl.kernel(
        body,
        out_type=tuple(out_type),
        mesh=mesh,
        compiler_params=pltpu.CompilerParams(needs_layout_passes=False,
                                             use_tc_tiling_on_sc=False),
        scratch_types=scratch,
    )(vt, ex, src, dst)
    return res if isinstance(res, tuple) else (res,)


def _final(x, parts, wat, ba_row, alpha):
    n = x.shape[0]
    t = 'word' if n == _NN['word'] else ('topic' if n == _NN['topic']
                                         else 'doc')
    bn = _BN[t]
    npad = _NPAD[t]
    cblk = npad // bn
    grid = (n // bn,)

    def mkmap(cid):
        return functools.partial(
            lambda i, cid, cblk: (cid * cblk + i, 0), cid=cid, cblk=cblk)

    ins = [x]
    in_specs = [pl.BlockSpec((bn, _D), lambda i: (i, 0))]
    counts = []
    for num_arrs, den in parts:
        cnt = 0
        for arr, nc in num_arrs:
            for cid in range(2):
                ins.append(arr)
                in_specs.append(pl.BlockSpec((bn, nc), mkmap(cid)))
                cnt += 1
        for cid in range(2):
            ins.append(den)
            in_specs.append(pl.BlockSpec((bn, _EW), mkmap(cid)))
            cnt += 1
        counts.append(cnt)
    ins += [wat, ba_row, alpha]
    in_specs += [pl.BlockSpec((_D, _D), lambda i: (0, 0)),
                 pl.BlockSpec((1, _D), lambda i: (0, 0)),
                 pl.BlockSpec((1, 1), lambda i: (0, 0),
                              memory_space=pltpu.SMEM)]

    def body(x_ref, *refs):
        pos = 0
        msgs = []
        for (num_arrs, den), cnt in zip(parts, counts):
            group = refs[pos:pos + cnt]
            pos += cnt
            gi = 0
            num_cols = []
            for arr, nc in num_arrs:
                num_cols.append(group[gi][...] + group[gi + 1][...])
                gi += 2
            num = jnp.concatenate(num_cols, axis=1)
            den_v = (group[gi][...] + group[gi + 1][...])[:, :_H]
            den_rep = jnp.repeat(den_v, _DK, axis=1)
            h = num / jnp.maximum(den_rep, 1e-30)
            msgs.append(jax.nn.relu(h))
        wat_ref, ba_ref, al_ref = refs[pos], refs[pos + 1], refs[pos + 2]
        out_ref = refs[pos + 3]
        msg = msgs[0]
        for m in msgs[1:]:
            msg = msg + m
        msg = msg * (1.0 / len(msgs))
        al = al_ref[0, 0]
        trans = jnp.dot(msg, wat_ref[...],
                        preferred_element_type=jnp.float32) + ba_ref[...]
        out_ref[...] = trans * al + x_ref[...] * (1.0 - al)

    return pl.pallas_call(
        body,
        grid=grid,
        in_specs=in_specs,
        out_specs=pl.BlockSpec((bn, _D), lambda i: (i, 0)),
        out_shape=jax.ShapeDtypeStruct((n, _D), jnp.float32),
    )(*ins)


def kernel(x_word, x_topic, x_doc, src_ww, dst_ww, src_wt, dst_wt, src_wd,
           dst_wd, src_tt, dst_tt, src_td, dst_td, Wk, Wq, Wv, Wa, bk, bq,
           bv, ba, skip, rel_pri, rel_att, rel_msg):
    f32 = jnp.float32
    x = {'word': x_word, 'topic': x_topic, 'doc': x_doc}
    src = {'ww': src_ww, 'wt': src_wt, 'wd': src_wd, 'tt': src_tt,
           'td': src_td}
    dst = {'ww': dst_ww, 'wt': dst_wt, 'wd': dst_wd, 'tt': dst_tt,
           'td': dst_td}
    src = {k: v.astype(jnp.int32) for k, v in src.items()}
    dst = {k: v.astype(jnp.int32) for k, v in dst.items()}

    mk, ck, mv, cv, wqs, bqs = _combine_weights(
        Wk, Wq, Wv, bk, bq, bv, rel_att, rel_pri, rel_msg)

    wc_word = jnp.concatenate(
        [wqs[0], mk[0], mk[1], mk[2], mv[1], mv[2], mv[0]], axis=1)
    bc_word = jnp.concatenate(
        [bqs[0], ck[0], ck[1], ck[2], cv[1], cv[2], cv[0]]).reshape(1, -1)
    qs_w, kt_ww, kt_wt, kt_wd, vt_wt, vt_wd, vw0, vw1, vw2, vw3 = _prep_type(
        x['word'], wc_word, bc_word, [128, 128, 128, 128, 128, 128,
                                      32, 32, 32, 32])
    wc_topic = jnp.concatenate([wqs[1], mk[3], mk[4], mv[3], mv[4]], axis=1)
    bc_topic = jnp.concatenate(
        [bqs[1], ck[3], ck[4], cv[3], cv[4]]).reshape(1, -1)
    qs_t, kt_tt, kt_td, vt_tt, vt_td = _prep_type(
        x['topic'], wc_topic, bc_topic, [128, 128, 128, 128, 128])
    (qs_d,) = _prep_type(x['doc'], wqs[2], bqs[2].reshape(1, -1), [128])

    qs = {'word': qs_w, 'topic': qs_t, 'doc': qs_d}
    kt = {'ww': kt_ww, 'wt': kt_wt, 'wd': kt_wd, 'tt': kt_tt, 'td': kt_td}
    vt = {'wt': vt_wt, 'wd': vt_wd, 'tt': vt_tt, 'td': vt_td}
    vtww = [vw0, vw1, vw2, vw3]

    tables = [(kt[et], qs[d], src[et], dst[et], ne)
              for et, s, d, ne in _REL]
    atts = _sc_att(tables)
    att = {et: a for (et, _, _, _), a in zip(_REL, atts)}

    ex = {et: _max_exp(att[et]) for et, *_ in _REL}

    npw, npt, npd = _NPAD['word'], _NPAD['topic'], _NPAD['doc']
    ww_num = []
    for g in range(4):
        (numg,) = _sc_agg(vtww[g], ex['ww'], src['ww'], dst['ww'],
                          ne=262144, npad=npw, ncols=32, head_lo=2 * g,
                          with_den=False)
        ww_num.append(numg)
    (ww_den,) = _sc_agg(vtww[0], ex['ww'], src['ww'], dst['ww'],
                        ne=262144, npad=npw, ncols=16, head_lo=0,
                        with_den=True, do_num=False)
    num_wt, den_wt = _sc_agg(vt['wt'], ex['wt'], src['wt'], dst['wt'],
                             ne=65536, npad=npt, ncols=128, head_lo=0,
                             with_den=True)
    num_tt, den_tt = _sc_agg(vt['tt'], ex['tt'], src['tt'], dst['tt'],
                             ne=32768, npad=npt, ncols=128, head_lo=0,
                             with_den=True)
    num_wd, den_wd = _sc_agg(vt['wd'], ex['wd'], src['wd'], dst['wd'],
                             ne=131072, npad=npd, ncols=128, head_lo=0,
                             with_den=True, blk_sz=32)
    num_td, den_td = _sc_agg(vt['td'], ex['td'], src['td'], dst['td'],
                             ne=65536, npad=npd, ncols=128, head_lo=0,
                             with_den=True, blk_sz=32)

    alpha = jax.nn.sigmoid(skip).astype(f32)
    wat = jnp.transpose(Wa, (0, 2, 1))
    out_w = _final(x['word'],
                   [([(g, 32) for g in ww_num], ww_den)],
                   wat[0], ba[0].reshape(1, _D),
                   alpha[0].reshape(1, 1))
    out_t = _final(x['topic'],
                   [([(num_wt, 128)], den_wt), ([(num_tt, 128)], den_tt)],
                   wat[1], ba[1].reshape(1, _D),
                   alpha[1].reshape(1, 1))
    out_d = _final(x['doc'],
                   [([(num_wd, 128)], den_wd), ([(num_td, 128)], den_td)],
                   wat[2], ba[2].reshape(1, _D),
                   alpha[2].reshape(1, 1))
    return (out_w, out_t, out_d)
